# baseline (speedup 1.0000x reference)
import functools

import jax
import jax.numpy as jnp
from jax.experimental import pallas as pl
from jax.experimental.pallas import tpu as pltpu

M_TILE = 256


def _router_kernel(x_ref, w_ref, b_ref, emb_ref, out_ref, h_ref, embn_ref):
    m = pl.program_id(0)
    steps = pl.num_programs(0)

    @pl.when(m == 0)
    def _():
        emb_t = emb_ref[...].T
        inv = jax.lax.rsqrt(
            jnp.maximum(jnp.sum(emb_t * emb_t, axis=0, keepdims=True), 1e-24)
        )
        embn_ref[...] = (emb_t * inv).astype(jnp.bfloat16)

    @pl.when(m < steps - 1)
    def _():
        h = (
            jnp.dot(x_ref[...], w_ref[...], preferred_element_type=jnp.float32)
            + b_ref[...]
        ).astype(jnp.bfloat16)
        h_ref[m % 2, :, :] = h

    @pl.when(m > 0)
    def _():
        out_ref[...] = jax.lax.dot_general(
            h_ref[(m - 1) % 2, :, :], embn_ref[...],
            dimension_numbers=(((1,), (0,)), ((), ())),
            preferred_element_type=jnp.float32,
        )


@functools.partial(jax.jit, static_argnums=())
def kernel(x, W, b, neuron_emb):
    Bb, S, D = x.shape
    N, d_space = neuron_emb.shape
    M = Bb * S
    x2 = x.reshape(M, D)
    b2 = b.reshape(1, d_space)

    n_tiles = M // M_TILE
    grid = (n_tiles + 1,)
    out = pl.pallas_call(
        _router_kernel,
        grid=grid,
        in_specs=[
            pl.BlockSpec((M_TILE, D), lambda m: (jnp.minimum(m, n_tiles - 1), 0)),
            pl.BlockSpec((D, d_space), lambda m: (0, 0)),
            pl.BlockSpec((1, d_space), lambda m: (0, 0)),
            pl.BlockSpec((N, d_space), lambda m: (0, 0)),
        ],
        out_specs=pl.BlockSpec(
            (M_TILE, N), lambda m: (jnp.maximum(m - 1, 0), 0)
        ),
        out_shape=jax.ShapeDtypeStruct((M, N), jnp.float32),
        scratch_shapes=[
            pltpu.VMEM((2, M_TILE, d_space), jnp.bfloat16),
            pltpu.VMEM((d_space, N), jnp.bfloat16),
        ],
        compiler_params=pltpu.CompilerParams(
            dimension_semantics=("arbitrary",),
        ),
    )(x2, W, b2, neuron_emb)
    return out.reshape(Bb, S, N)


# speedup vs baseline: 1.0299x; 1.0299x over previous
import functools

import jax
import jax.numpy as jnp
from jax.experimental import pallas as pl
from jax.experimental.pallas import tpu as pltpu

M_TILE = 512


def _router_kernel(x_ref, w_ref, b_ref, emb_ref, out_ref, h_ref, embn_ref):
    m = pl.program_id(0)
    steps = pl.num_programs(0)

    @pl.when(m == 0)
    def _():
        emb_t = emb_ref[...].T
        inv = jax.lax.rsqrt(
            jnp.maximum(jnp.sum(emb_t * emb_t, axis=0, keepdims=True), 1e-24)
        )
        embn_ref[...] = (emb_t * inv).astype(jnp.bfloat16)

    @pl.when(m < steps - 1)
    def _():
        h = (
            jnp.dot(x_ref[...], w_ref[...], preferred_element_type=jnp.float32)
            + b_ref[...]
        ).astype(jnp.bfloat16)
        h_ref[m % 2, :, :] = h

    @pl.when(m > 0)
    def _():
        out_ref[...] = jax.lax.dot_general(
            h_ref[(m - 1) % 2, :, :], embn_ref[...],
            dimension_numbers=(((1,), (0,)), ((), ())),
            preferred_element_type=jnp.float32,
        )


@functools.partial(jax.jit, static_argnums=())
def kernel(x, W, b, neuron_emb):
    Bb, S, D = x.shape
    N, d_space = neuron_emb.shape
    M = Bb * S
    x2 = x.reshape(M, D)
    b2 = b.reshape(1, d_space)

    n_tiles = M // M_TILE
    grid = (n_tiles + 1,)
    out = pl.pallas_call(
        _router_kernel,
        grid=grid,
        in_specs=[
            pl.BlockSpec((M_TILE, D), lambda m: (jnp.minimum(m, n_tiles - 1), 0)),
            pl.BlockSpec((D, d_space), lambda m: (0, 0)),
            pl.BlockSpec((1, d_space), lambda m: (0, 0)),
            pl.BlockSpec((N, d_space), lambda m: (0, 0)),
        ],
        out_specs=pl.BlockSpec(
            (M_TILE, N), lambda m: (jnp.maximum(m - 1, 0), 0)
        ),
        out_shape=jax.ShapeDtypeStruct((M, N), jnp.float32),
        scratch_shapes=[
            pltpu.VMEM((2, M_TILE, d_space), jnp.bfloat16),
            pltpu.VMEM((d_space, N), jnp.bfloat16),
        ],
        compiler_params=pltpu.CompilerParams(
            dimension_semantics=("arbitrary",),
        ),
    )(x2, W, b2, neuron_emb)
    return out.reshape(Bb, S, N)


# speedup vs baseline: 1.0325x; 1.0026x over previous
import functools

import jax
import jax.numpy as jnp
from jax.experimental import pallas as pl
from jax.experimental.pallas import tpu as pltpu

M_TILE = 512


def _router_kernel(x_ref, w_ref, b_ref, emb_ref, out_ref, h_ref, embn_ref):
    m = pl.program_id(0)

    @pl.when(m == 0)
    def _():
        emb_t = emb_ref[...].T
        inv = jax.lax.rsqrt(
            jnp.maximum(jnp.sum(emb_t * emb_t, axis=0, keepdims=True), 1e-24)
        )
        embn_ref[...] = (emb_t * inv).astype(jnp.bfloat16)

    h_ref[m % 2, :, :] = (
        jnp.dot(x_ref[...], w_ref[...], preferred_element_type=jnp.float32)
        + b_ref[...]
    ).astype(jnp.bfloat16)

    @pl.when(m > 0)
    def _():
        out_ref[...] = jax.lax.dot_general(
            h_ref[(m - 1) % 2, :, :], embn_ref[...],
            dimension_numbers=(((1,), (0,)), ((), ())),
            preferred_element_type=jnp.float32,
        )


@functools.partial(jax.jit, static_argnums=())
def kernel(x, W, b, neuron_emb):
    Bb, S, D = x.shape
    N, d_space = neuron_emb.shape
    M = Bb * S
    x2 = x.reshape(M, D)
    b2 = b.reshape(1, d_space)

    n_tiles = M // M_TILE
    grid = (n_tiles + 1,)
    out = pl.pallas_call(
        _router_kernel,
        grid=grid,
        in_specs=[
            pl.BlockSpec((M_TILE, D), lambda m: (jnp.minimum(m, n_tiles - 1), 0)),
            pl.BlockSpec((D, d_space), lambda m: (0, 0)),
            pl.BlockSpec((1, d_space), lambda m: (0, 0)),
            pl.BlockSpec((N, d_space), lambda m: (0, 0)),
        ],
        out_specs=pl.BlockSpec(
            (M_TILE, N), lambda m: (jnp.maximum(m - 1, 0), 0)
        ),
        out_shape=jax.ShapeDtypeStruct((M, N), jnp.float32),
        scratch_shapes=[
            pltpu.VMEM((2, M_TILE, d_space), jnp.bfloat16),
            pltpu.VMEM((d_space, N), jnp.bfloat16),
        ],
        compiler_params=pltpu.CompilerParams(
            dimension_semantics=("arbitrary",),
        ),
    )(x2, W, b2, neuron_emb)
    return out.reshape(Bb, S, N)
